# baseline (speedup 1.0000x reference)
import jax
import jax.numpy as jnp
from jax.experimental import pallas as pl
from jax.experimental.pallas import tpu as pltpu

Q = 1024
D = 64
N = 100000
B = 2048
NFULL = 48
NTAIL = N - NFULL * B
TPAD = 1792
K = 5
LANES = 128
QS = 64
BIG = 1e30


def _insert(dist_slab, lab_row, width, vrefs, lrefs):
    for qb in range(Q // QS):
        qlo = qb * QS
        vals = [vrefs[k][qlo:qlo + QS, :] for k in range(K)]
        labs = [lrefs[k][qlo:qlo + QS, :] for k in range(K)]
        for c in range(width // LANES):
            clo = c * LANES
            v = dist_slab[qlo:qlo + QS, clo:clo + LANES]
            l = jnp.broadcast_to(lab_row[:, clo:clo + LANES], (QS, LANES))
            for k in range(K):
                cond = v < vals[k]
                nv = jnp.minimum(v, vals[k])
                xv = jnp.maximum(v, vals[k])
                nl = jnp.where(cond, l, labs[k])
                xl = jnp.where(cond, labs[k], l)
                vals[k] = nv
                labs[k] = nl
                v = xv
                l = xl
        for k in range(K):
            vrefs[k][qlo:qlo + QS, :] = vals[k]
            lrefs[k][qlo:qlo + QS, :] = labs[k]


def _knn_body(x_ref, data_ref, dsq_ref, labels_ref,
              tdata_ref, tdsq_ref, tlabels_ref, out_ref, dist_ref, *regs):
    vrefs = regs[:K]
    lrefs = regs[K:]
    n = pl.program_id(0)

    @pl.when(n == 0)
    def _init():
        for k in range(K):
            vrefs[k][:] = jnp.full((Q, LANES), BIG, jnp.float32)
            lrefs[k][:] = jnp.zeros((Q, LANES), jnp.int32)

    x = x_ref[:]
    xa = x * (-2.0)
    xsq = jnp.sum(x * x, axis=1, keepdims=True)

    H = B // 2
    for h in range(2):
        lo = h * H
        cross = jax.lax.dot_general(
            xa, data_ref[lo:lo + H, :], (((1,), (1,)), ((), ())),
            precision=jax.lax.Precision.DEFAULT,
            preferred_element_type=jnp.float32)
        dist_ref[:, lo:lo + H] = (xsq + dsq_ref[0, :, lo:lo + H]) + cross
        _insert(dist_ref[:, lo:lo + H], labels_ref[0, :, lo:lo + H],
                H, vrefs, lrefs)

    @pl.when(n == NFULL - 1)
    def _tail_and_finish():
        cross_t = jax.lax.dot_general(
            xa, tdata_ref[:], (((1,), (1,)), ((), ())),
            precision=jax.lax.Precision.DEFAULT,
            preferred_element_type=jnp.float32)
        dist_ref[:, :TPAD] = (xsq + tdsq_ref[:]) + cross_t
        _insert(dist_ref[:, :TPAD], tlabels_ref[:], TPAD, vrefs, lrefs)

        V = jnp.concatenate([vrefs[k][:] for k in range(K)], axis=1)
        L = jnp.concatenate([lrefs[k][:] for k in range(K)], axis=1)
        cols = jax.lax.broadcasted_iota(jnp.int32, (Q, K * LANES), 1)
        knn_labs = []
        for _ in range(K):
            m = jnp.min(V, axis=1, keepdims=True)
            pos = jnp.min(jnp.where(V == m, cols, jnp.int32(1 << 30)),
                          axis=1, keepdims=True)
            sel = cols == pos
            knn_labs.append(jnp.sum(jnp.where(sel, L, 0), axis=1, keepdims=True))
            V = jnp.where(sel, BIG, V)
        best = jnp.full((Q, 1), -1, jnp.int32)
        pred = jnp.zeros((Q, 1), jnp.int32)
        for i in range(K):
            cnt = knn_labs[0] * 0
            for j in range(K):
                cnt = cnt + (knn_labs[i] == knn_labs[j]).astype(jnp.int32)
            score = cnt * 16384 - knn_labs[i]
            take = score > best
            best = jnp.where(take, score, best)
            pred = jnp.where(take, knn_labs[i], pred)
        out_ref[:] = pred


def kernel(x, data, labels):
    nfr = NFULL * B
    dsq = jnp.sum(data * data, axis=1)
    dsq_m = dsq[:nfr].reshape(NFULL, 1, B)
    labels_m = labels[:nfr].reshape(NFULL, 1, B)
    tdata = jnp.concatenate(
        [data[nfr:], jnp.zeros((TPAD - NTAIL, D), data.dtype)], axis=0)
    tdsq = jnp.concatenate(
        [dsq[nfr:], jnp.full((TPAD - NTAIL,), 1e10, jnp.float32)]).reshape(1, TPAD)
    tlabels = jnp.concatenate(
        [labels[nfr:], jnp.zeros((TPAD - NTAIL,), labels.dtype)]).reshape(1, TPAD)

    preds = pl.pallas_call(
        _knn_body,
        grid=(NFULL,),
        in_specs=[
            pl.BlockSpec((Q, D), lambda n: (0, 0)),
            pl.BlockSpec((B, D), lambda n: (n, 0)),
            pl.BlockSpec((1, 1, B), lambda n: (n, 0, 0)),
            pl.BlockSpec((1, 1, B), lambda n: (n, 0, 0)),
            pl.BlockSpec((TPAD, D), lambda n: (0, 0)),
            pl.BlockSpec((1, TPAD), lambda n: (0, 0)),
            pl.BlockSpec((1, TPAD), lambda n: (0, 0)),
        ],
        out_specs=pl.BlockSpec((Q, 1), lambda n: (0, 0)),
        out_shape=jax.ShapeDtypeStruct((Q, 1), jnp.int32),
        scratch_shapes=(
            [pltpu.VMEM((Q, B), jnp.float32)]
            + [pltpu.VMEM((Q, LANES), jnp.float32) for _ in range(K)]
            + [pltpu.VMEM((Q, LANES), jnp.int32) for _ in range(K)]
        ),
        compiler_params=pltpu.CompilerParams(
            dimension_semantics=("arbitrary",)),
    )(x, data, dsq_m, labels_m, tdata, tdsq, tlabels)
    return preds.reshape(Q)


# speedup vs baseline: 1.0038x; 1.0038x over previous
import jax
import jax.numpy as jnp
from jax.experimental import pallas as pl
from jax.experimental.pallas import tpu as pltpu

Q = 1024
D = 64
N = 100000
B = 2048
NFULL = 48
NTAIL = N - NFULL * B
TPAD = 1792
K = 5
LANES = 128
QS = 64
BIG = 1e30


def _insert(dist_slab, lab_row, width, vrefs, lrefs):
    for qb in range(Q // QS):
        qlo = qb * QS
        vals = [vrefs[k][qlo:qlo + QS, :] for k in range(K)]
        labs = [lrefs[k][qlo:qlo + QS, :] for k in range(K)]
        for c in range(width // LANES):
            clo = c * LANES
            v = dist_slab[qlo:qlo + QS, clo:clo + LANES]
            l = jnp.broadcast_to(lab_row[:, clo:clo + LANES], (QS, LANES))
            for k in range(K):
                cond = v < vals[k]
                nv = jnp.minimum(v, vals[k])
                xv = jnp.maximum(v, vals[k])
                nl = jnp.where(cond, l, labs[k])
                xl = jnp.where(cond, labs[k], l)
                vals[k] = nv
                labs[k] = nl
                v = xv
                l = xl
        for k in range(K):
            vrefs[k][qlo:qlo + QS, :] = vals[k]
            lrefs[k][qlo:qlo + QS, :] = labs[k]


def _knn_body(x_ref, data_ref, dsq_ref, labels_ref,
              tdata_ref, tdsq_ref, tlabels_ref, out_ref, dist_ref, *regs):
    vrefs = regs[:K]
    lrefs = regs[K:]
    n = pl.program_id(0)

    @pl.when(n == 0)
    def _init():
        for k in range(K):
            vrefs[k][:] = jnp.full((Q, LANES), BIG, jnp.float32)
            lrefs[k][:] = jnp.zeros((Q, LANES), jnp.int32)

    x = x_ref[:]
    xa = x * (-2.0)
    xsq = jnp.sum(x * x, axis=1, keepdims=True)

    cross = jax.lax.dot_general(
        xa, data_ref[:], (((1,), (1,)), ((), ())),
        precision=jax.lax.Precision.DEFAULT,
        preferred_element_type=jnp.float32)
    dist_ref[:, :B] = (xsq + dsq_ref[0]) + cross
    _insert(dist_ref[:, :B], labels_ref[0], B, vrefs, lrefs)

    @pl.when(n == NFULL - 1)
    def _tail_and_finish():
        cross_t = jax.lax.dot_general(
            xa, tdata_ref[:], (((1,), (1,)), ((), ())),
            precision=jax.lax.Precision.DEFAULT,
            preferred_element_type=jnp.float32)
        dist_ref[:, :TPAD] = (xsq + tdsq_ref[:]) + cross_t
        _insert(dist_ref[:, :TPAD], tlabels_ref[:], TPAD, vrefs, lrefs)

        V = jnp.concatenate([vrefs[k][:] for k in range(K)], axis=1)
        L = jnp.concatenate([lrefs[k][:] for k in range(K)], axis=1)
        cols = jax.lax.broadcasted_iota(jnp.int32, (Q, K * LANES), 1)
        knn_labs = []
        for _ in range(K):
            m = jnp.min(V, axis=1, keepdims=True)
            pos = jnp.min(jnp.where(V == m, cols, jnp.int32(1 << 30)),
                          axis=1, keepdims=True)
            sel = cols == pos
            knn_labs.append(jnp.sum(jnp.where(sel, L, 0), axis=1, keepdims=True))
            V = jnp.where(sel, BIG, V)
        best = jnp.full((Q, 1), -1, jnp.int32)
        pred = jnp.zeros((Q, 1), jnp.int32)
        for i in range(K):
            cnt = knn_labs[0] * 0
            for j in range(K):
                cnt = cnt + (knn_labs[i] == knn_labs[j]).astype(jnp.int32)
            score = cnt * 16384 - knn_labs[i]
            take = score > best
            best = jnp.where(take, score, best)
            pred = jnp.where(take, knn_labs[i], pred)
        out_ref[:] = pred


def kernel(x, data, labels):
    nfr = NFULL * B
    dsq = jnp.sum(data * data, axis=1)
    dsq_m = dsq[:nfr].reshape(NFULL, 1, B)
    labels_m = labels[:nfr].reshape(NFULL, 1, B)
    tdata = jnp.concatenate(
        [data[nfr:], jnp.zeros((TPAD - NTAIL, D), data.dtype)], axis=0)
    tdsq = jnp.concatenate(
        [dsq[nfr:], jnp.full((TPAD - NTAIL,), 1e10, jnp.float32)]).reshape(1, TPAD)
    tlabels = jnp.concatenate(
        [labels[nfr:], jnp.zeros((TPAD - NTAIL,), labels.dtype)]).reshape(1, TPAD)

    preds = pl.pallas_call(
        _knn_body,
        grid=(NFULL,),
        in_specs=[
            pl.BlockSpec((Q, D), lambda n: (0, 0)),
            pl.BlockSpec((B, D), lambda n: (n, 0)),
            pl.BlockSpec((1, 1, B), lambda n: (n, 0, 0)),
            pl.BlockSpec((1, 1, B), lambda n: (n, 0, 0)),
            pl.BlockSpec((TPAD, D), lambda n: (0, 0)),
            pl.BlockSpec((1, TPAD), lambda n: (0, 0)),
            pl.BlockSpec((1, TPAD), lambda n: (0, 0)),
        ],
        out_specs=pl.BlockSpec((Q, 1), lambda n: (0, 0)),
        out_shape=jax.ShapeDtypeStruct((Q, 1), jnp.int32),
        scratch_shapes=(
            [pltpu.VMEM((Q, B), jnp.float32)]
            + [pltpu.VMEM((Q, LANES), jnp.float32) for _ in range(K)]
            + [pltpu.VMEM((Q, LANES), jnp.int32) for _ in range(K)]
        ),
        compiler_params=pltpu.CompilerParams(
            dimension_semantics=("arbitrary",)),
    )(x, data, dsq_m, labels_m, tdata, tdsq, tlabels)
    return preds.reshape(Q)


# speedup vs baseline: 1.0791x; 1.0749x over previous
import jax
import jax.numpy as jnp
from jax.experimental import pallas as pl
from jax.experimental.pallas import tpu as pltpu

Q = 1024
D = 64
N = 100000
B = 2048
NFULL = 48
NTAIL = N - NFULL * B
TPAD = 1792
K = 5
LANES = 128
QS = 64
BIG = 1e30


def _dist(xa, xsq, d, dsq_row):
    cross = jax.lax.dot_general(
        xa, d, (((1,), (1,)), ((), ())),
        precision=jax.lax.Precision.DEFAULT,
        preferred_element_type=jnp.float32)
    return (xsq + dsq_row) + cross


def _insert_vals(dist_slab, width, vrefs):
    for qb in range(Q // QS):
        qlo = qb * QS
        vals = [vrefs[k][qlo:qlo + QS, :] for k in range(K)]
        for c in range(width // LANES):
            v = dist_slab[qlo:qlo + QS, c * LANES:(c + 1) * LANES]
            for k in range(K):
                nv = jnp.minimum(v, vals[k])
                v = jnp.maximum(v, vals[k])
                vals[k] = nv
        for k in range(K):
            vrefs[k][qlo:qlo + QS, :] = vals[k]


def _vote(dist_slab, lab_col, width, thr, votes_ref):
    mask = (dist_slab <= thr).astype(jnp.bfloat16)
    cls = jax.lax.broadcasted_iota(jnp.int32, (width, LANES), 1)
    onehot = (lab_col == cls).astype(jnp.bfloat16)
    votes_ref[:] += jax.lax.dot_general(
        mask, onehot, (((1,), (0,)), ((), ())),
        preferred_element_type=jnp.float32)


def _knn_body(x_ref, data_ref, dsq_ref, labels_ref,
              tdata_ref, tdsq_ref, tlabels_ref, out_ref,
              dist_ref, thr_ref, votes_ref, *vrefs):
    p = pl.program_id(0)
    n = pl.program_id(1)

    @pl.when(jnp.logical_and(p == 0, n == 0))
    def _init():
        for k in range(K):
            vrefs[k][:] = jnp.full((Q, LANES), BIG, jnp.float32)
        votes_ref[:] = jnp.zeros((Q, LANES), jnp.float32)

    x = x_ref[:]
    xa = x * (-2.0)
    xsq = jnp.sum(x * x, axis=1, keepdims=True)

    @pl.when(p == 0)
    def _sweep_values():
        dist_ref[:, :B] = _dist(xa, xsq, data_ref[:], dsq_ref[0])
        _insert_vals(dist_ref[:, :B], B, vrefs)

        @pl.when(n == NFULL - 1)
        def _tail_and_threshold():
            dist_ref[:, :TPAD] = _dist(xa, xsq, tdata_ref[:], tdsq_ref[:])
            _insert_vals(dist_ref[:, :TPAD], TPAD, vrefs)

            V = jnp.concatenate([vrefs[k][:] for k in range(K)], axis=1)
            cols = jax.lax.broadcasted_iota(jnp.int32, (Q, K * LANES), 1)
            m = None
            for _ in range(K):
                m = jnp.min(V, axis=1, keepdims=True)
                pos = jnp.min(jnp.where(V == m, cols, jnp.int32(1 << 30)),
                              axis=1, keepdims=True)
                V = jnp.where(cols == pos, BIG, V)
            thr_ref[:] = jnp.broadcast_to(m, (Q, LANES))

    @pl.when(p == 1)
    def _sweep_votes():
        thr = thr_ref[:, :1]
        dist_ref[:, :B] = _dist(xa, xsq, data_ref[:], dsq_ref[0])
        _vote(dist_ref[:, :B], labels_ref[:], B, thr, votes_ref)

        @pl.when(n == NFULL - 1)
        def _tail_and_predict():
            dist_ref[:, :TPAD] = _dist(xa, xsq, tdata_ref[:], tdsq_ref[:])
            _vote(dist_ref[:, :TPAD], tlabels_ref[:], TPAD, thr, votes_ref)

            votes = votes_ref[:]
            cls_q = jax.lax.broadcasted_iota(jnp.int32, (Q, LANES), 1)
            mx = jnp.max(votes, axis=1, keepdims=True)
            pred = jnp.min(jnp.where(votes == mx, cls_q, jnp.int32(1 << 30)),
                           axis=1, keepdims=True)
            out_ref[:] = pred


def kernel(x, data, labels):
    nfr = NFULL * B
    dsq = jnp.sum(data * data, axis=1)
    dsq_m = dsq[:nfr].reshape(NFULL, 1, B)
    labels_m = labels[:nfr].reshape(nfr, 1)
    tdata = jnp.concatenate(
        [data[nfr:], jnp.zeros((TPAD - NTAIL, D), data.dtype)], axis=0)
    tdsq = jnp.concatenate(
        [dsq[nfr:], jnp.full((TPAD - NTAIL,), 1e10, jnp.float32)]).reshape(1, TPAD)
    tlabels = jnp.concatenate(
        [labels[nfr:], jnp.full((TPAD - NTAIL,), -1, labels.dtype)]).reshape(TPAD, 1)

    preds = pl.pallas_call(
        _knn_body,
        grid=(2, NFULL),
        in_specs=[
            pl.BlockSpec((Q, D), lambda p, n: (0, 0)),
            pl.BlockSpec((B, D), lambda p, n: (n, 0)),
            pl.BlockSpec((1, 1, B), lambda p, n: (n, 0, 0)),
            pl.BlockSpec((B, 1), lambda p, n: (n, 0)),
            pl.BlockSpec((TPAD, D), lambda p, n: (0, 0)),
            pl.BlockSpec((1, TPAD), lambda p, n: (0, 0)),
            pl.BlockSpec((TPAD, 1), lambda p, n: (0, 0)),
        ],
        out_specs=pl.BlockSpec((Q, 1), lambda p, n: (0, 0)),
        out_shape=jax.ShapeDtypeStruct((Q, 1), jnp.int32),
        scratch_shapes=(
            [pltpu.VMEM((Q, B), jnp.float32),
             pltpu.VMEM((Q, LANES), jnp.float32),
             pltpu.VMEM((Q, LANES), jnp.float32)]
            + [pltpu.VMEM((Q, LANES), jnp.float32) for _ in range(K)]
        ),
        compiler_params=pltpu.CompilerParams(
            dimension_semantics=("arbitrary", "arbitrary")),
    )(x, data, dsq_m, labels_m, tdata, tdsq, tlabels)
    return preds.reshape(Q)
